# baseline (speedup 1.0000x reference)
import functools

import jax
import jax.numpy as jnp
from jax import lax
from jax.experimental import pallas as pl
from jax.experimental.pallas import tpu as pltpu
from jax.experimental.pallas import tpu_sc as plsc

VOCAB = 1000000
DIM = 32
B = 4096
L = 200

NC = 2
NS = 16
NW = NC * NS
SEQ_W = B // NW


def _body(tok_hbm, table_hbm, out_hbm, idx_v, buf0, buf1, gsem0, gsem1,
          wsem0, wsem1):
    wid = lax.axis_index("s") * NC + lax.axis_index("c")
    bufs = (buf0, buf1)
    gsems = (gsem0, gsem1)
    wsems = (wsem0, wsem1)
    pltpu.sync_copy(tok_hbm.at[pl.ds(wid * SEQ_W, SEQ_W)], idx_v)

    def gather(c, b):
        pltpu.async_copy(table_hbm.at[idx_v.at[c]], bufs[b], gsems[b])

    def write(c, b):
        pltpu.async_copy(bufs[b], out_hbm.at[wid * SEQ_W + c], wsems[b])

    gather(0, 0)
    gather(1, 1)

    @pl.loop(0, SEQ_W, step=2)
    def _pair(g):
        for b in range(2):
            c = g + b
            pltpu.make_async_copy(table_hbm.at[idx_v.at[c]], bufs[b],
                                  gsems[b]).wait()
            write(c, b)

            @pl.when(c + 2 < SEQ_W)
            def _():
                pltpu.make_async_copy(bufs[b], out_hbm.at[0], wsems[b]).wait()
                gather(c + 2, b)

    for b in range(2):
        pltpu.make_async_copy(bufs[b], out_hbm.at[0], wsems[b]).wait()


@functools.partial(jax.jit, static_argnames=())
def kernel(tokens, word_weight):
    grid_kernel = pl.kernel(
        _body,
        out_type=jax.ShapeDtypeStruct((B, L, DIM), jnp.float32),
        mesh=plsc.VectorSubcoreMesh(core_axis_name="c", subcore_axis_name="s"),
        scratch_types=[
            pltpu.VMEM((SEQ_W, L), jnp.int32),
            pltpu.VMEM((L, DIM), jnp.float32),
            pltpu.VMEM((L, DIM), jnp.float32),
            pltpu.SemaphoreType.DMA,
            pltpu.SemaphoreType.DMA,
            pltpu.SemaphoreType.DMA,
            pltpu.SemaphoreType.DMA,
        ],
        compiler_params=pltpu.CompilerParams(use_tc_tiling_on_sc=False),
    )
    return grid_kernel(tokens.astype(jnp.int32), word_weight)


# speedup vs baseline: 1.0246x; 1.0246x over previous
import functools

import jax
import jax.numpy as jnp
from jax import lax
from jax.experimental import pallas as pl
from jax.experimental.pallas import tpu as pltpu
from jax.experimental.pallas import tpu_sc as plsc

VOCAB = 1000000
DIM = 32
B = 4096
L = 200

NC = 2
NS = 16
NW = NC * NS
SEQ_W = B // NW
CH = 4
NCH = SEQ_W // CH
ROWS = CH * L


def _body(tok_hbm, table_hbm, out_hbm, idx_v, buf0, buf1, gsem0, gsem1,
          wsem0, wsem1):
    wid = lax.axis_index("s") * NC + lax.axis_index("c")
    base = wid * SEQ_W * L
    bufs = (buf0, buf1)
    gsems = (gsem0, gsem1)
    wsems = (wsem0, wsem1)
    pltpu.sync_copy(tok_hbm.at[pl.ds(base, SEQ_W * L)], idx_v)

    def gather(c, b):
        pltpu.async_copy(table_hbm.at[idx_v.at[pl.ds(c * ROWS, ROWS)]],
                         bufs[b], gsems[b])

    def write(c, b):
        pltpu.async_copy(bufs[b], out_hbm.at[pl.ds(base + c * ROWS, ROWS)],
                         wsems[b])

    gather(0, 0)
    gather(1, 1)

    @pl.loop(0, NCH, step=2)
    def _pair(g):
        for b in range(2):
            c = g + b
            pltpu.make_async_copy(
                table_hbm.at[idx_v.at[pl.ds(c * ROWS, ROWS)]], bufs[b],
                gsems[b]).wait()
            write(c, b)

            @pl.when(c + 2 < NCH)
            def _():
                pltpu.make_async_copy(bufs[b], out_hbm.at[pl.ds(0, ROWS)],
                                      wsems[b]).wait()
                gather(c + 2, b)

    for b in range(2):
        pltpu.make_async_copy(bufs[b], out_hbm.at[pl.ds(0, ROWS)],
                              wsems[b]).wait()


@functools.partial(jax.jit, static_argnames=())
def kernel(tokens, word_weight):
    grid_kernel = pl.kernel(
        _body,
        out_type=jax.ShapeDtypeStruct((B * L, DIM), jnp.float32),
        mesh=plsc.VectorSubcoreMesh(core_axis_name="c", subcore_axis_name="s"),
        scratch_types=[
            pltpu.VMEM((SEQ_W * L,), jnp.int32),
            pltpu.VMEM((ROWS, DIM), jnp.float32),
            pltpu.VMEM((ROWS, DIM), jnp.float32),
            pltpu.SemaphoreType.DMA,
            pltpu.SemaphoreType.DMA,
            pltpu.SemaphoreType.DMA,
            pltpu.SemaphoreType.DMA,
        ],
        compiler_params=pltpu.CompilerParams(use_tc_tiling_on_sc=False),
    )
    flat = grid_kernel(tokens.reshape(-1).astype(jnp.int32), word_weight)
    return flat.reshape(B, L, DIM)
